# baseline (speedup 1.0000x reference)
import jax
import jax.numpy as jnp
from jax import lax
from jax.experimental import pallas as pl
from jax.experimental.pallas import tpu as pltpu
from jax.experimental.pallas import tpu_sc as plsc

_G = 256
_VOX = 0.8
_NBOX = 20

_NW = 32
_CELLS = (_G * _G) // _NW
_KC = 8
_SC_CH = 16
_SC_C0 = 128 - _SC_CH

_TC_CHUNK = 16


def _sc_body(a_hbm, o_hbm, pa_hbm, po_hbm,
             ba0, ba1, bo0, bo1, acc_a, acc_o,
             sem_a0, sem_a1, sem_o0, sem_o1):
    wid = lax.axis_index("s") * 2 + lax.axis_index("c")
    cell0 = wid * _CELLS
    nchunks = _SC_CH // _KC

    for b in range(2):
        bufs_a = (ba0, ba1)
        bufs_o = (bo0, bo1)
        sems_a = (sem_a0, sem_a1)
        sems_o = (sem_o0, sem_o1)

        pltpu.async_copy(
            a_hbm.at[b, pl.ds(_SC_C0, _KC), pl.ds(cell0, _CELLS)],
            bufs_a[0], sems_a[0])
        pltpu.async_copy(
            o_hbm.at[b, pl.ds(_SC_C0, _KC), pl.ds(cell0, _CELLS)],
            bufs_o[0], sems_o[0])

        for k in range(nchunks):
            cur = k % 2
            nxt = (k + 1) % 2
            if k + 1 < nchunks:
                c0 = _SC_C0 + (k + 1) * _KC
                pltpu.async_copy(
                    a_hbm.at[b, pl.ds(c0, _KC), pl.ds(cell0, _CELLS)],
                    bufs_a[nxt], sems_a[nxt])
                pltpu.async_copy(
                    o_hbm.at[b, pl.ds(c0, _KC), pl.ds(cell0, _CELLS)],
                    bufs_o[nxt], sems_o[nxt])
            pltpu.make_async_copy(
                a_hbm.at[b, pl.ds(_SC_C0, _KC), pl.ds(cell0, _CELLS)],
                bufs_a[cur], sems_a[cur]).wait()
            pltpu.make_async_copy(
                o_hbm.at[b, pl.ds(_SC_C0, _KC), pl.ds(cell0, _CELLS)],
                bufs_o[cur], sems_o[cur]).wait()

            ba = bufs_a[cur]
            bo = bufs_o[cur]
            first = (k == 0)

            def mk_accum(ba=ba, bo=bo, first=first):
                @plsc.parallel_loop(0, _CELLS, step=16, unroll=4)
                def _(i):
                    sl = pl.ds(i, 16)
                    va = (ba[0, sl] + ba[1, sl]) + (ba[2, sl] + ba[3, sl])
                    va = va + ((ba[4, sl] + ba[5, sl]) + (ba[6, sl] + ba[7, sl]))
                    vo = (bo[0, sl] + bo[1, sl]) + (bo[2, sl] + bo[3, sl])
                    vo = vo + ((bo[4, sl] + bo[5, sl]) + (bo[6, sl] + bo[7, sl]))
                    if first:
                        acc_a[sl] = va
                        acc_o[sl] = vo
                    else:
                        acc_a[sl] = acc_a[sl] + va
                        acc_o[sl] = acc_o[sl] + vo

            mk_accum()

        pltpu.sync_copy(acc_a, pa_hbm.at[b, pl.ds(cell0, _CELLS)])
        pltpu.sync_copy(acc_o, po_hbm.at[b, pl.ds(cell0, _CELLS)])


def _sc_partial_maps(a3, o3):
    mesh = plsc.VectorSubcoreMesh(core_axis_name="c", subcore_axis_name="s")
    f = pl.kernel(
        _sc_body,
        out_type=[
            jax.ShapeDtypeStruct((2, _G * _G), jnp.float32),
            jax.ShapeDtypeStruct((2, _G * _G), jnp.float32),
        ],
        mesh=mesh,
        scratch_types=[
            pltpu.VMEM((_KC, _CELLS), jnp.float32),
            pltpu.VMEM((_KC, _CELLS), jnp.float32),
            pltpu.VMEM((_KC, _CELLS), jnp.float32),
            pltpu.VMEM((_KC, _CELLS), jnp.float32),
            pltpu.VMEM((_CELLS,), jnp.float32),
            pltpu.VMEM((_CELLS,), jnp.float32),
            pltpu.SemaphoreType.DMA,
            pltpu.SemaphoreType.DMA,
            pltpu.SemaphoreType.DMA,
            pltpu.SemaphoreType.DMA,
        ],
    )
    return f(a3, o3)


def _tc_partial_kernel(a_ref, o_ref, o_last_ref, ta_ref, to_ref, acc_a, acc_o):
    k = pl.program_id(1)
    nk = pl.num_programs(1)

    a_sum = jnp.sum(a_ref[0], axis=0)

    @pl.when(k == 0)
    def _():
        acc_a[...] = a_sum
        acc_o[...] = jnp.sum(o_ref[0, 1:], axis=0)

    @pl.when(k > 0)
    def _():
        acc_a[...] += a_sum
        acc_o[...] += jnp.sum(o_ref[0], axis=0)

    @pl.when(k == nk - 1)
    def _():
        ta_ref[0] = acc_a[...]
        to_ref[0] = acc_o[...] + o_last_ref[0, 0]


def _tc_partial_maps(added_points, original_points):
    B, C, H, W = added_points.shape
    nk = _SC_C0 // _TC_CHUNK
    return pl.pallas_call(
        _tc_partial_kernel,
        grid=(B, nk),
        in_specs=[
            pl.BlockSpec((1, _TC_CHUNK, H, W), lambda b, k: (b, k, 0, 0)),
            pl.BlockSpec((1, _TC_CHUNK, H, W), lambda b, k: (b, k, 0, 0)),
            pl.BlockSpec((1, 1, H, W), lambda b, k: (b, C, 0, 0)),
        ],
        out_specs=[
            pl.BlockSpec((1, H, W), lambda b, k: (b, 0, 0)),
            pl.BlockSpec((1, H, W), lambda b, k: (b, 0, 0)),
        ],
        out_shape=[
            jax.ShapeDtypeStruct((B, H, W), jnp.float32),
            jax.ShapeDtypeStruct((B, H, W), jnp.float32),
        ],
        scratch_shapes=[
            pltpu.VMEM((H, W), jnp.float32),
            pltpu.VMEM((H, W), jnp.float32),
        ],
    )(added_points, original_points, original_points)


def _combine_kernel(boxes_ref, ta_ref, pa_ref, to_ref, po_ref,
                    inter_ref, union_ref):
    pred_occ = (ta_ref[0] + pa_ref[0]) != 0.0
    orig_occ = (to_ref[0] + po_ref[0]) != 0.0
    ii = jax.lax.broadcasted_iota(jnp.int32, (_G, _G), 0)
    jj = jax.lax.broadcasted_iota(jnp.int32, (_G, _G), 1)
    x = (ii.astype(jnp.float32) - _G / 2.0) * _VOX
    y = (jj.astype(jnp.float32) - _G / 2.0) * _VOX
    boxes = boxes_ref[0]
    mask = jnp.zeros((_G, _G), dtype=jnp.bool_)
    for t in range(_NBOX):
        cx = boxes[t, 0]
        cy = boxes[t, 1]
        cz = boxes[t, 2]
        dx = boxes[t, 3]
        dy = boxes[t, 4]
        dz = boxes[t, 5]
        hd = boxes[t, 6]
        sx = x - cx
        sy = y - cy
        cth = jnp.cos(hd)
        sth = jnp.sin(hd)
        lx = sx * cth + sy * sth
        ly = sy * cth - sx * sth
        zin = jnp.abs(_VOX - cz) <= dz * 0.5
        inb = (jnp.abs(lx) <= dx * 0.5) & (jnp.abs(ly) <= dy * 0.5) & zin
        mask = mask | inb
    p = pred_occ & mask
    o = orig_occ & mask
    inter = jnp.sum(jnp.where(p & o, 1.0, 0.0))
    union = jnp.sum(jnp.where(p | o, 1.0, 0.0))
    inter_ref[0] = jnp.full((8, 128), inter, jnp.float32)
    union_ref[0] = jnp.full((8, 128), union, jnp.float32)


def _combine(boxes_p, ta, pa, to, po):
    B = ta.shape[0]
    return pl.pallas_call(
        _combine_kernel,
        grid=(B,),
        in_specs=[
            pl.BlockSpec((1, 24, 128), lambda b: (b, 0, 0)),
            pl.BlockSpec((1, _G, _G), lambda b: (b, 0, 0)),
            pl.BlockSpec((1, _G, _G), lambda b: (b, 0, 0)),
            pl.BlockSpec((1, _G, _G), lambda b: (b, 0, 0)),
            pl.BlockSpec((1, _G, _G), lambda b: (b, 0, 0)),
        ],
        out_specs=[
            pl.BlockSpec((1, 8, 128), lambda b: (b, 0, 0)),
            pl.BlockSpec((1, 8, 128), lambda b: (b, 0, 0)),
        ],
        out_shape=[
            jax.ShapeDtypeStruct((B, 8, 128), jnp.float32),
            jax.ShapeDtypeStruct((B, 8, 128), jnp.float32),
        ],
    )(boxes_p, ta, pa, to, po)


def kernel(added_points, original_points, boxes):
    B, C, H, W = added_points.shape
    a3 = added_points.reshape(B, C, H * W)
    o3 = original_points.reshape(B, C + 1, H * W)

    pa, po = _sc_partial_maps(a3, o3)
    ta, to = _tc_partial_maps(added_points, original_points)

    boxes_p = jnp.zeros((B, 24, 128), jnp.float32).at[:, :_NBOX, :7].set(boxes)
    inter, union = _combine(
        boxes_p, ta, pa.reshape(B, H, W), to, po.reshape(B, H, W))
    iou = inter[:, 0, 0] / jnp.maximum(union[:, 0, 0], 1.0)
    return jnp.mean(iou)


# speedup vs baseline: 11.1340x; 11.1340x over previous
import jax
import jax.numpy as jnp
from jax.experimental import pallas as pl
from jax.experimental.pallas import tpu as pltpu

_GRID = 256
_VOX = 0.8
_NBOX = 20
_CHUNK = 16


def _loss_kernel(boxes_ref, a_ref, o_ref, o_last_ref, inter_ref, union_ref,
                 acc_a, acc_o, mask_s):
    k = pl.program_id(1)
    nk = pl.num_programs(1)

    a_sum = jnp.sum(a_ref[0], axis=0)

    @pl.when(k == 0)
    def _():
        acc_a[...] = a_sum
        acc_o[...] = jnp.sum(o_ref[0, 1:], axis=0)

    @pl.when(k > 0)
    def _():
        acc_a[...] += a_sum
        acc_o[...] += jnp.sum(o_ref[0], axis=0)

    @pl.when(k == 1)
    def _():
        ii = jax.lax.broadcasted_iota(jnp.int32, (_GRID, _GRID), 0)
        jj = jax.lax.broadcasted_iota(jnp.int32, (_GRID, _GRID), 1)
        x = (ii.astype(jnp.float32) - _GRID / 2.0) * _VOX
        y = (jj.astype(jnp.float32) - _GRID / 2.0) * _VOX
        boxes = boxes_ref[0]
        mask = jnp.zeros((_GRID, _GRID), dtype=jnp.bool_)
        for t in range(_NBOX):
            cx = boxes[t, 0]
            cy = boxes[t, 1]
            cz = boxes[t, 2]
            dx = boxes[t, 3]
            dy = boxes[t, 4]
            dz = boxes[t, 5]
            hd = boxes[t, 6]
            sx = x - cx
            sy = y - cy
            cth = jnp.cos(hd)
            sth = jnp.sin(hd)
            lx = sx * cth + sy * sth
            ly = sy * cth - sx * sth
            zin = jnp.abs(_VOX - cz) <= dz * 0.5
            inb = (jnp.abs(lx) <= dx * 0.5) & (jnp.abs(ly) <= dy * 0.5) & zin
            mask = mask | inb
        mask_s[...] = jnp.where(mask, 1.0, 0.0)

    @pl.when(k == nk - 1)
    def _():
        pred_occ = acc_a[...] != 0.0
        orig_occ = (acc_o[...] + o_last_ref[0, 0]) != 0.0
        m = mask_s[...] != 0.0
        p = pred_occ & m
        o = orig_occ & m
        inter = jnp.sum(jnp.where(p & o, 1.0, 0.0))
        union = jnp.sum(jnp.where(p | o, 1.0, 0.0))
        inter_ref[0] = jnp.full((8, 128), inter, jnp.float32)
        union_ref[0] = jnp.full((8, 128), union, jnp.float32)


def kernel(added_points, original_points, boxes):
    B, C, H, W = added_points.shape
    boxes_p = jnp.zeros((B, 24, 128), jnp.float32).at[:, :_NBOX, :7].set(boxes)
    nk = C // _CHUNK
    inter, union = pl.pallas_call(
        _loss_kernel,
        grid=(B, nk),
        in_specs=[
            pl.BlockSpec((1, 24, 128), lambda b, k: (b, 0, 0)),
            pl.BlockSpec((1, _CHUNK, H, W), lambda b, k: (b, k, 0, 0)),
            pl.BlockSpec((1, _CHUNK, H, W), lambda b, k: (b, k, 0, 0)),
            pl.BlockSpec((1, 1, H, W), lambda b, k: (b, C, 0, 0)),
        ],
        out_specs=[
            pl.BlockSpec((1, 8, 128), lambda b, k: (b, 0, 0)),
            pl.BlockSpec((1, 8, 128), lambda b, k: (b, 0, 0)),
        ],
        out_shape=[
            jax.ShapeDtypeStruct((B, 8, 128), jnp.float32),
            jax.ShapeDtypeStruct((B, 8, 128), jnp.float32),
        ],
        scratch_shapes=[
            pltpu.VMEM((H, W), jnp.float32),
            pltpu.VMEM((H, W), jnp.float32),
            pltpu.VMEM((H, W), jnp.float32),
        ],
    )(boxes_p, added_points, original_points, original_points)
    iou = inter[:, 0, 0] / jnp.maximum(union[:, 0, 0], 1.0)
    return jnp.mean(iou)


# speedup vs baseline: 11.1379x; 1.0003x over previous
import jax
import jax.numpy as jnp
from jax.experimental import pallas as pl
from jax.experimental.pallas import tpu as pltpu

_GRID = 256
_VOX = 0.8
_NBOX = 20
_CHUNK = 16


def _loss_kernel(boxes_ref, a_ref, o_ref, o_last_ref, inter_ref, union_ref,
                 acc_a, acc_o, mask_s):
    k = pl.program_id(1)
    nk = pl.num_programs(1)

    a_sum = jnp.sum(a_ref[0], axis=0)

    @pl.when(k == 0)
    def _():
        acc_a[...] = a_sum
        acc_o[...] = jnp.sum(o_ref[0, 1:], axis=0)

    @pl.when(k > 0)
    def _():
        acc_a[...] += a_sum
        acc_o[...] += jnp.sum(o_ref[0], axis=0)

    @pl.when(k == 1)
    def _():
        ii = jax.lax.broadcasted_iota(jnp.int32, (_GRID, _GRID), 0)
        jj = jax.lax.broadcasted_iota(jnp.int32, (_GRID, _GRID), 1)
        x = (ii.astype(jnp.float32) - _GRID / 2.0) * _VOX
        y = (jj.astype(jnp.float32) - _GRID / 2.0) * _VOX
        boxes = boxes_ref[0]
        mask = jnp.zeros((_GRID, _GRID), dtype=jnp.bool_)
        for t in range(_NBOX):
            cx = boxes[t, 0]
            cy = boxes[t, 1]
            cz = boxes[t, 2]
            dx = boxes[t, 3]
            dy = boxes[t, 4]
            dz = boxes[t, 5]
            hd = boxes[t, 6]
            sx = x - cx
            sy = y - cy
            cth = jnp.cos(hd)
            sth = jnp.sin(hd)
            lx = sx * cth + sy * sth
            ly = sy * cth - sx * sth
            zin = jnp.abs(_VOX - cz) <= dz * 0.5
            inb = (jnp.abs(lx) <= dx * 0.5) & (jnp.abs(ly) <= dy * 0.5) & zin
            mask = mask | inb
        mask_s[...] = jnp.where(mask, 1.0, 0.0)

    @pl.when(k == nk - 1)
    def _():
        pred_occ = acc_a[...] != 0.0
        orig_occ = (acc_o[...] + o_last_ref[0, 0]) != 0.0
        m = mask_s[...] != 0.0
        p = pred_occ & m
        o = orig_occ & m
        pi = jnp.where(p & o, 1.0, 0.0)
        pu = jnp.where(p | o, 1.0, 0.0)
        ones_c = jnp.ones((_GRID, 2), jnp.float32)
        ones_r = jnp.ones((2, _GRID), jnp.float32)
        si = jnp.dot(ones_r, jnp.dot(pi, ones_c),
                     preferred_element_type=jnp.float32)
        su = jnp.dot(ones_r, jnp.dot(pu, ones_c),
                     preferred_element_type=jnp.float32)
        inter_ref[0] = jnp.full((8, 128), si[0, 0], jnp.float32)
        union_ref[0] = jnp.full((8, 128), su[0, 0], jnp.float32)


def kernel(added_points, original_points, boxes):
    B, C, H, W = added_points.shape
    boxes_p = jnp.zeros((B, 24, 128), jnp.float32).at[:, :_NBOX, :7].set(boxes)
    nk = C // _CHUNK
    inter, union = pl.pallas_call(
        _loss_kernel,
        grid=(B, nk),
        in_specs=[
            pl.BlockSpec((1, 24, 128), lambda b, k: (b, 0, 0)),
            pl.BlockSpec((1, _CHUNK, H, W), lambda b, k: (b, k, 0, 0)),
            pl.BlockSpec((1, _CHUNK, H, W), lambda b, k: (b, k, 0, 0)),
            pl.BlockSpec((1, 1, H, W), lambda b, k: (b, C, 0, 0)),
        ],
        out_specs=[
            pl.BlockSpec((1, 8, 128), lambda b, k: (b, 0, 0)),
            pl.BlockSpec((1, 8, 128), lambda b, k: (b, 0, 0)),
        ],
        out_shape=[
            jax.ShapeDtypeStruct((B, 8, 128), jnp.float32),
            jax.ShapeDtypeStruct((B, 8, 128), jnp.float32),
        ],
        scratch_shapes=[
            pltpu.VMEM((H, W), jnp.float32),
            pltpu.VMEM((H, W), jnp.float32),
            pltpu.VMEM((H, W), jnp.float32),
        ],
    )(boxes_p, added_points, original_points, original_points)
    iou = inter[:, 0, 0] / jnp.maximum(union[:, 0, 0], 1.0)
    return jnp.mean(iou)


# speedup vs baseline: 11.9312x; 1.0712x over previous
import jax
import jax.numpy as jnp
from jax.experimental import pallas as pl
from jax.experimental.pallas import tpu as pltpu

_GRID = 256
_VOX = 0.8
_NBOX = 20
_CHUNK = 32


def _loss_kernel(boxes_ref, a_ref, o_ref, o_last_ref, inter_ref, union_ref,
                 acc_a, acc_o, mask_s):
    k = pl.program_id(1)
    nk = pl.num_programs(1)

    a_sum = jnp.sum(a_ref[0], axis=0)

    @pl.when(k == 0)
    def _():
        acc_a[...] = a_sum
        acc_o[...] = jnp.sum(o_ref[0, 1:], axis=0)

    @pl.when(k > 0)
    def _():
        acc_a[...] += a_sum
        acc_o[...] += jnp.sum(o_ref[0], axis=0)

    @pl.when(k == 1)
    def _():
        ii = jax.lax.broadcasted_iota(jnp.int32, (_GRID, _GRID), 0)
        jj = jax.lax.broadcasted_iota(jnp.int32, (_GRID, _GRID), 1)
        x = (ii.astype(jnp.float32) - _GRID / 2.0) * _VOX
        y = (jj.astype(jnp.float32) - _GRID / 2.0) * _VOX
        boxes = boxes_ref[0]
        mask = jnp.zeros((_GRID, _GRID), dtype=jnp.bool_)
        for t in range(_NBOX):
            cx = boxes[t, 0]
            cy = boxes[t, 1]
            cz = boxes[t, 2]
            dx = boxes[t, 3]
            dy = boxes[t, 4]
            dz = boxes[t, 5]
            hd = boxes[t, 6]
            sx = x - cx
            sy = y - cy
            cth = jnp.cos(hd)
            sth = jnp.sin(hd)
            lx = sx * cth + sy * sth
            ly = sy * cth - sx * sth
            zin = jnp.abs(_VOX - cz) <= dz * 0.5
            inb = (jnp.abs(lx) <= dx * 0.5) & (jnp.abs(ly) <= dy * 0.5) & zin
            mask = mask | inb
        mask_s[...] = jnp.where(mask, 1.0, 0.0)

    @pl.when(k == nk - 1)
    def _():
        pred_occ = acc_a[...] != 0.0
        orig_occ = (acc_o[...] + o_last_ref[0, 0]) != 0.0
        m = mask_s[...] != 0.0
        p = pred_occ & m
        o = orig_occ & m
        pi = jnp.where(p & o, 1.0, 0.0)
        pu = jnp.where(p | o, 1.0, 0.0)
        ones_c = jnp.ones((_GRID, 2), jnp.float32)
        ones_r = jnp.ones((2, _GRID), jnp.float32)
        si = jnp.dot(ones_r, jnp.dot(pi, ones_c),
                     preferred_element_type=jnp.float32)
        su = jnp.dot(ones_r, jnp.dot(pu, ones_c),
                     preferred_element_type=jnp.float32)
        inter_ref[0] = jnp.full((8, 128), si[0, 0], jnp.float32)
        union_ref[0] = jnp.full((8, 128), su[0, 0], jnp.float32)


def kernel(added_points, original_points, boxes):
    B, C, H, W = added_points.shape
    boxes_p = jnp.zeros((B, 24, 128), jnp.float32).at[:, :_NBOX, :7].set(boxes)
    nk = C // _CHUNK
    inter, union = pl.pallas_call(
        _loss_kernel,
        grid=(B, nk),
        in_specs=[
            pl.BlockSpec((1, 24, 128), lambda b, k: (b, 0, 0)),
            pl.BlockSpec((1, _CHUNK, H, W), lambda b, k: (b, k, 0, 0)),
            pl.BlockSpec((1, _CHUNK, H, W), lambda b, k: (b, k, 0, 0)),
            pl.BlockSpec((1, 1, H, W), lambda b, k: (b, C, 0, 0)),
        ],
        out_specs=[
            pl.BlockSpec((1, 8, 128), lambda b, k: (b, 0, 0)),
            pl.BlockSpec((1, 8, 128), lambda b, k: (b, 0, 0)),
        ],
        out_shape=[
            jax.ShapeDtypeStruct((B, 8, 128), jnp.float32),
            jax.ShapeDtypeStruct((B, 8, 128), jnp.float32),
        ],
        scratch_shapes=[
            pltpu.VMEM((H, W), jnp.float32),
            pltpu.VMEM((H, W), jnp.float32),
            pltpu.VMEM((H, W), jnp.float32),
        ],
    )(boxes_p, added_points, original_points, original_points)
    iou = inter[:, 0, 0] / jnp.maximum(union[:, 0, 0], 1.0)
    return jnp.mean(iou)
